# baseline (speedup 1.0000x reference)
import numpy as np
import jax
import jax.numpy as jnp
from jax.experimental import pallas as pl
from jax.sharding import Mesh, PartitionSpec as P

try:
    from jax.experimental.shard_map import shard_map as _shard_map
except ImportError:
    _shard_map = jax.shard_map

_B, _N = 16, 128
_DN, _DE = 512, 128
_ATOM_VOCABS = (65, 6, 12, 8, 7, 3, 6, 6, 3)
_BOND_VOCABS = (5, 3, 3, 7, 7)
_R_BLK = 16384


def _offsets(vocabs):
    offs, o = [], 0
    for v in vocabs:
        offs.append(o)
        o += v
    return offs


def _seg_consts(vocabs, ncols, klanes):
    s = np.zeros((ncols, klanes), np.float32)
    c = np.ones((1, klanes), np.float32)
    for f, (off, v) in enumerate(zip(_offsets(vocabs), vocabs)):
        s[f, off:off + v] = 1.0
        c[0, off:off + v] = off - np.arange(off, off + v)
    return s, c


def _hilo(x):
    hi = x.astype(jnp.bfloat16)
    lo = (x - hi.astype(jnp.float32)).astype(jnp.bfloat16)
    return hi, lo


def _body(stk_ref, s_ref, c_ref, tcat_ref, w_ref, out_ref):
    stk = stk_ref[...]
    bmat = jnp.dot(stk, s_ref[...], preferred_element_type=jnp.float32)
    ohf = ((bmat + c_ref[...]) == 0).astype(jnp.bfloat16)
    mm = jnp.dot(ohf, tcat_ref[...], preferred_element_type=jnp.float32)
    mm2 = jnp.dot(stk, w_ref[...], preferred_element_type=jnp.float32)
    out_ref[...] = mm + mm2


def _pad_cat(tables, rows):
    cat = jnp.concatenate(tables, axis=0)
    cat = jnp.pad(cat, ((0, rows - cat.shape[0]), (0, 0)))
    return cat.astype(jnp.bfloat16)


def _emb_call(stk, s_c, c_c, tcat, w, r_blk, dout):
    r = stk.shape[0]
    r_blk = min(r_blk, r)
    ncols = stk.shape[1]
    return pl.pallas_call(
        _body,
        grid=(r // r_blk,),
        in_specs=[pl.BlockSpec((r_blk, ncols), lambda i: (i, 0)),
                  pl.BlockSpec((ncols, 128), lambda i: (0, 0)),
                  pl.BlockSpec((1, 128), lambda i: (0, 0)),
                  pl.BlockSpec((128, dout), lambda i: (0, 0)),
                  pl.BlockSpec((ncols, dout), lambda i: (0, 0))],
        out_specs=pl.BlockSpec((r_blk, dout), lambda i: (i, 0)),
        out_shape=jax.ShapeDtypeStruct((r, dout), jnp.float32),
    )(stk, s_c, c_c, tcat, w)


def kernel(atomic_number, formal_charge, degree, explicit_valence,
           implicit_valence, aromatic, hybridization, total_num_H, is_in_ring,
           bond_type, conjugated, ring, stereo, shortest_path, position,
           relative_distance, w_atomic_number, w_formal_charge, w_degree,
           w_explicit_valence, w_implicit_valence, w_aromatic, w_hybridization,
           w_total_num_H, w_is_in_ring, w_bond_type, w_conjugated, w_ring,
           w_stereo, w_shortest_path, pos_w, rel_w):
    bs_np, bc_np = _seg_consts(_BOND_VOCABS, 8, 128)
    as_np, ac_np = _seg_consts(_ATOM_VOCABS, 24, 128)

    devs = jax.devices()
    ndev = 2 if len(devs) >= 2 and _B % 2 == 0 else 1
    mesh = Mesh(np.array(devs[:ndev]), ("x",))

    def shard_fn(a0, a1, a2, a3, a4, a5, a6, a7, a8, b0, b1, b2, b3, b4,
                 pos, rel, a_tcat, b_tcat, posw, relw):
        bsh = a0.shape[0]
        bn = bsh * _N
        rows = bn * _N

        bs_c = jnp.asarray(bs_np, jnp.bfloat16)
        bc_c = jnp.asarray(bc_np, jnp.float32)
        as_c = jnp.asarray(as_np, jnp.bfloat16)
        ac_c = jnp.asarray(ac_np, jnp.float32)

        r_hi, r_lo = _hilo(rel)
        bstk = jnp.stack(
            [b0.astype(jnp.bfloat16), b1.astype(jnp.bfloat16),
             b2.astype(jnp.bfloat16), b3.astype(jnp.bfloat16),
             b4.astype(jnp.bfloat16), r_hi, r_hi, r_lo],
            axis=-1).reshape(rows, 8)
        w_hi, w_lo = _hilo(relw)
        w8 = jnp.concatenate(
            [jnp.zeros((5, _DE), jnp.bfloat16), w_hi, w_lo, w_hi], axis=0)
        bond = _emb_call(bstk, bs_c, bc_c, b_tcat, w8, _R_BLK, _DE)

        p_hi, p_lo = _hilo(pos)
        astk = jnp.concatenate(
            [a0.astype(jnp.bfloat16)[..., None],
             a1.astype(jnp.bfloat16)[..., None],
             a2.astype(jnp.bfloat16)[..., None],
             a3.astype(jnp.bfloat16)[..., None],
             a4.astype(jnp.bfloat16)[..., None],
             a5.astype(jnp.bfloat16)[..., None],
             a6.astype(jnp.bfloat16)[..., None],
             a7.astype(jnp.bfloat16)[..., None],
             a8.astype(jnp.bfloat16)[..., None],
             p_hi, p_hi, p_lo, jnp.zeros((bsh, _N, 6), jnp.bfloat16)],
            axis=-1).reshape(bn, 24)
        pw_hi, pw_lo = _hilo(posw)
        w24 = jnp.concatenate(
            [jnp.zeros((9, _DN), jnp.bfloat16), pw_hi, pw_lo, pw_hi,
             jnp.zeros((6, _DN), jnp.bfloat16)], axis=0)
        atom = _emb_call(astk, as_c, ac_c, a_tcat, w24, _R_BLK, _DN)

        return (atom.reshape(bsh, _N, _DN),
                bond.reshape(bsh, _N, _N, _DE))

    atom_tcat = _pad_cat((w_atomic_number, w_formal_charge, w_degree,
                          w_explicit_valence, w_implicit_valence, w_aromatic,
                          w_hybridization, w_total_num_H, w_is_in_ring), 128)
    bond_tcat = _pad_cat((w_bond_type, w_conjugated, w_ring, w_stereo,
                          w_shortest_path), 128)

    sh = P("x") if ndev > 1 else P()
    rep = P()
    atom_out, bond_out = _shard_map(
        shard_fn, mesh=mesh, check_rep=False,
        in_specs=(sh,) * 16 + (rep,) * 4,
        out_specs=(sh, sh),
    )(atomic_number, formal_charge, degree, explicit_valence, implicit_valence,
      aromatic, hybridization, total_num_H, is_in_ring, bond_type, conjugated,
      ring, stereo, shortest_path, position, relative_distance, atom_tcat,
      bond_tcat, pos_w, rel_w)
    return atom_out, bond_out
